# Initial kernel scaffold; baseline (speedup 1.0000x reference)
import jax
import jax.numpy as jnp
from jax.experimental import pallas as pl


def kernel(input, weight):
    raise NotImplementedError("write your pallas kernel here")


# speedup vs baseline: 2.9668x; 2.9668x over previous
import functools

import jax
import jax.numpy as jnp
from jax import lax
from jax.experimental import pallas as pl
from jax.experimental.pallas import tpu as pltpu
from jax.experimental.pallas import tpu_sc as plsc

D = 128
CHUNK = 128
NC, NS = 2, 16
NW = NC * NS


def _emb_body(n_chunks_per_w, idx_hbm, w_hbm, out_hbm, idx_v, rows_v, sem):
    wid = lax.axis_index("s") * NC + lax.axis_index("c")
    base_c = wid * n_chunks_per_w
    pltpu.sync_copy(idx_hbm.at[wid], idx_v)

    def body(g, carry):
        pltpu.async_copy(w_hbm.at[idx_v.at[g]], rows_v, sem).wait()
        pltpu.sync_copy(rows_v, out_hbm.at[pl.ds((base_c + g) * CHUNK, CHUNK)])
        return carry

    lax.fori_loop(0, n_chunks_per_w, body, 0)


@jax.jit
def kernel(input, weight):
    S0, S1 = input.shape
    B = S0 * S1
    n_chunks = B // CHUNK
    n_chunks_per_w = n_chunks // NW
    idx = input.reshape(NW, n_chunks_per_w, CHUNK).astype(jnp.int32)

    mesh = plsc.VectorSubcoreMesh(core_axis_name="c", subcore_axis_name="s")
    k = pl.kernel(
        functools.partial(_emb_body, n_chunks_per_w),
        mesh=mesh,
        out_type=jax.ShapeDtypeStruct((B, D), jnp.float32),
        scratch_types=[
            pltpu.VMEM((n_chunks_per_w, CHUNK), jnp.int32),
            pltpu.VMEM((CHUNK, D), jnp.float32),
            pltpu.SemaphoreType.DMA,
        ],
    )
    out = k(idx, weight)
    return out.reshape(S0, S1, D)


# speedup vs baseline: 3.3402x; 1.1258x over previous
import functools

import jax
import jax.numpy as jnp
from jax import lax
from jax.experimental import pallas as pl
from jax.experimental.pallas import tpu as pltpu
from jax.experimental.pallas import tpu_sc as plsc

D = 128
CHUNK = 128
NC, NS = 2, 16
NW = NC * NS


NBUF = 5


def _emb_body(n_chunks_per_w, idx_hbm, w_hbm, out_hbm, idx_v, rows_v, *sems):
    wid = lax.axis_index("s") * NC + lax.axis_index("c")
    base_c = wid * n_chunks_per_w
    pltpu.sync_copy(idx_hbm.at[wid], idx_v)

    for b in range(NBUF):
        pltpu.async_copy(w_hbm.at[idx_v.at[b]], rows_v.at[b], sems[b])

    def body(t, carry):
        g0 = t * NBUF
        for b in range(NBUF):
            g = g0 + b
            pltpu.make_async_copy(w_hbm.at[idx_v.at[0]], rows_v.at[b],
                                  sems[b]).wait()
            pltpu.sync_copy(rows_v.at[b],
                            out_hbm.at[pl.ds((base_c + g) * CHUNK, CHUNK)])
            nxt = g + NBUF

            @pl.when(nxt < n_chunks_per_w)
            def _():
                pltpu.async_copy(w_hbm.at[idx_v.at[nxt]], rows_v.at[b],
                                 sems[b])
        return carry

    lax.fori_loop(0, n_chunks_per_w // NBUF, body, 0)


@jax.jit
def kernel(input, weight):
    S0, S1 = input.shape
    B = S0 * S1
    n_chunks = B // CHUNK
    n_chunks_per_w = n_chunks // NW
    idx = input.reshape(NW, n_chunks_per_w, CHUNK).astype(jnp.int32)

    mesh = plsc.VectorSubcoreMesh(core_axis_name="c", subcore_axis_name="s")
    k = pl.kernel(
        functools.partial(_emb_body, n_chunks_per_w),
        mesh=mesh,
        out_type=jax.ShapeDtypeStruct((B, D), jnp.float32),
        scratch_types=[
            pltpu.VMEM((n_chunks_per_w, CHUNK), jnp.int32),
            pltpu.VMEM((NBUF, CHUNK, D), jnp.float32),
        ] + [pltpu.SemaphoreType.DMA] * NBUF,
    )
    out = k(idx, weight)
    return out.reshape(S0, S1, D)
